# baseline (speedup 1.0000x reference)
import jax
import jax.numpy as jnp
from jax import lax
from jax.experimental import pallas as pl
from jax.experimental.pallas import tpu as pltpu
from jax.experimental.pallas import tpu_sc as plsc

BATCH = 16384
EMB = 64
NROWS = 1000000
_RCH = 32768
_NCH = (NROWS + _RCH - 1) // _RCH
_LOG_RCH = _RCH.bit_length() - 1
_HMASK = _RCH // 2 - 1

_info = plsc.get_sparse_core_info()
_NC, _NS = _info.num_cores, _info.num_subcores
_NW = _NC * _NS
_BPW = BATCH // _NW


def _repack_body(src, dst):
    y = src[...].astype(jnp.bfloat16).T
    dst[...] = jnp.concatenate([y[: _RCH // 2], y[_RCH // 2:]], axis=1)


_repack = pl.pallas_call(
    _repack_body,
    grid=(_NCH,),
    in_specs=[pl.BlockSpec((EMB, _RCH), lambda j: (0, j))],
    out_specs=pl.BlockSpec((_RCH // 2, 2 * EMB), lambda j: (j, 0)),
    out_shape=jax.ShapeDtypeStruct((_NCH * _RCH // 2, 2 * EMB), jnp.bfloat16),
)


def _gather_body(idx_hbm, tbl, out_hbm, idx_v, idxh, vals, sem):
    wid = lax.axis_index("s") * _NC + lax.axis_index("c")
    base = wid * _BPW
    pltpu.sync_copy(idx_hbm.at[pl.ds(base, _BPW)], idx_v)

    def grp(g, c):
        r = idx_v[pl.ds(16 * g, 16)]
        idxh[pl.ds(16 * g, 16)] = (
            (r >> _LOG_RCH) << (_LOG_RCH - 1)) + (r & _HMASK)
        return c

    lax.fori_loop(0, _BPW // 16, grp, 0)
    pltpu.async_copy(tbl.at[idxh], vals, sem).wait()
    pltpu.sync_copy(vals, out_hbm.at[pl.ds(base, _BPW)])


_sc_gather = pl.kernel(
    _gather_body,
    out_type=jax.ShapeDtypeStruct((BATCH, 2 * EMB), jnp.bfloat16),
    mesh=plsc.VectorSubcoreMesh(core_axis_name="c", subcore_axis_name="s"),
    scratch_types=[
        pltpu.VMEM((_BPW,), jnp.int32),
        pltpu.VMEM((_BPW,), jnp.int32),
        pltpu.VMEM((_BPW, 2 * EMB), jnp.bfloat16),
        pltpu.SemaphoreType.DMA,
    ],
    compiler_params=pltpu.CompilerParams(use_tc_tiling_on_sc=False),
)


_BB = 2048


def _mlp_body(u2, i2, eu2, ei2, mu2, mi2,
              w1a, w1b, b1, w2, b2, w3, b3, wg, wh, bp, out):
    pu = ((u2[...] >> (_LOG_RCH - 1)) & 1) == 1
    pi = ((i2[...] >> (_LOG_RCH - 1)) & 1) == 1
    eu = jnp.where(pu, eu2[:, EMB:], eu2[:, :EMB]).astype(jnp.float32)
    mu = jnp.where(pu, mu2[:, EMB:], mu2[:, :EMB]).astype(jnp.float32)
    ei = jnp.where(pi, ei2[:, EMB:], ei2[:, :EMB]).astype(jnp.float32)
    mi = jnp.where(pi, mi2[:, EMB:], mi2[:, :EMB]).astype(jnp.float32)

    cdims = (((1,), (1,)), ((), ()))
    hp = jax.lax.Precision.HIGHEST
    gmf = eu * ei
    h = lax.dot_general(mu, w1a[...], cdims, precision=hp)
    h = h + lax.dot_general(mi, w1b[...], cdims, precision=hp)
    h = jnp.maximum(h + b1[...], 0.0)
    h = jnp.maximum(lax.dot_general(h, w2[...], cdims, precision=hp) + b2[...], 0.0)
    h = jnp.maximum(lax.dot_general(h, w3[...], cdims, precision=hp) + b3[...], 0.0)
    pred = jnp.sum(gmf * wg[...], axis=1) + jnp.sum(h * wh[...], axis=1)
    out[...] = pred + bp[0]


def _full(shape):
    nd = len(shape)
    return pl.BlockSpec(shape, lambda i: (0,) * nd)


def kernel(user, item, Ug, Ig, Um, Im, W1, b1, W2, b2, W3, b3, Wp, bp):
    user = user.astype(jnp.int32)
    item = item.astype(jnp.int32)

    eu2 = _sc_gather(user, _repack(Ug.T))
    ei2 = _sc_gather(item, _repack(Ig.T))
    mu2 = _sc_gather(user, _repack(Um.T))
    mi2 = _sc_gather(item, _repack(Im.T))

    w1a = W1[:, :EMB]
    w1b = W1[:, EMB:]
    h1 = W1.shape[0]
    h2 = W2.shape[0]
    h3 = W3.shape[0]
    wg = Wp[:, :EMB]
    wh = Wp[:, EMB:]

    grid = BATCH // _BB
    pair_spec = pl.BlockSpec((_BB, 2 * EMB), lambda i: (i, 0))
    idx_spec = pl.BlockSpec((_BB, 1), lambda i: (i, 0))
    out = pl.pallas_call(
        _mlp_body,
        grid=(grid,),
        in_specs=[
            idx_spec, idx_spec,
            pair_spec, pair_spec, pair_spec, pair_spec,
            _full((h1, EMB)), _full((h1, EMB)), _full((1, h1)),
            _full((h2, h1)), _full((1, h2)),
            _full((h3, h2)), _full((1, h3)),
            _full((1, EMB)), _full((1, h3)), _full((1,)),
        ],
        out_specs=pl.BlockSpec((_BB,), lambda i: (i,)),
        out_shape=jax.ShapeDtypeStruct((BATCH,), jnp.float32),
    )(user.reshape(BATCH, 1), item.reshape(BATCH, 1),
      eu2, ei2, mu2, mi2, w1a, w1b, b1.reshape(1, h1), W2,
      b2.reshape(1, h2), W3, b3.reshape(1, h3), wg, wh, bp)
    return out


# speedup vs baseline: 2.6179x; 2.6179x over previous
import jax
import jax.numpy as jnp
from jax import lax
from jax.experimental import pallas as pl
from jax.experimental.pallas import tpu as pltpu
from jax.experimental.pallas import tpu_sc as plsc

BATCH = 16384
EMB = 64
NROWS = 1000000
_RCH = 32768
_NCH = (NROWS + _RCH - 1) // _RCH
_LOG_RCH = _RCH.bit_length() - 1
_HMASK = _RCH // 2 - 1

_info = plsc.get_sparse_core_info()
_NC, _NS = _info.num_cores, _info.num_subcores
_NW = _NC * _NS
_BPW = BATCH // _NW


def _repack_body(src, dst):
    y = src[...].T
    dst[...] = jnp.concatenate([y[: _RCH // 2], y[_RCH // 2:]], axis=1)


_repack = pl.pallas_call(
    _repack_body,
    grid=(_NCH,),
    in_specs=[pl.BlockSpec((EMB, _RCH), lambda j: (0, j))],
    out_specs=pl.BlockSpec((_RCH // 2, 2 * EMB), lambda j: (j, 0)),
    out_shape=jax.ShapeDtypeStruct((_NCH * _RCH // 2, 2 * EMB), jnp.float32),
)


def _gather_body(idx_hbm, tbl, out_hbm, idx_v, idxh, vals, sem):
    wid = lax.axis_index("s") * _NC + lax.axis_index("c")
    base = wid * _BPW
    pltpu.sync_copy(idx_hbm.at[pl.ds(base, _BPW)], idx_v)

    def grp(g, c):
        r = idx_v[pl.ds(16 * g, 16)]
        idxh[pl.ds(16 * g, 16)] = (
            (r >> _LOG_RCH) << (_LOG_RCH - 1)) + (r & _HMASK)
        return c

    lax.fori_loop(0, _BPW // 16, grp, 0)
    pltpu.async_copy(tbl.at[idxh], vals, sem).wait()
    pltpu.sync_copy(vals, out_hbm.at[pl.ds(base, _BPW)])


_sc_gather = pl.kernel(
    _gather_body,
    out_type=jax.ShapeDtypeStruct((BATCH, 2 * EMB), jnp.float32),
    mesh=plsc.VectorSubcoreMesh(core_axis_name="c", subcore_axis_name="s"),
    scratch_types=[
        pltpu.VMEM((_BPW,), jnp.int32),
        pltpu.VMEM((_BPW,), jnp.int32),
        pltpu.VMEM((_BPW, 2 * EMB), jnp.float32),
        pltpu.SemaphoreType.DMA,
    ],
    compiler_params=pltpu.CompilerParams(use_tc_tiling_on_sc=False),
)


_BB = 2048


def _mlp_body(u2, i2, eu2, ei2, mu2, mi2,
              w1a, w1b, b1, w2, b2, w3, b3, wg, wh, bp, out):
    pu = ((u2[...] >> (_LOG_RCH - 1)) & 1) == 1
    pi = ((i2[...] >> (_LOG_RCH - 1)) & 1) == 1
    eu = jnp.where(pu, eu2[:, EMB:], eu2[:, :EMB])
    mu = jnp.where(pu, mu2[:, EMB:], mu2[:, :EMB])
    ei = jnp.where(pi, ei2[:, EMB:], ei2[:, :EMB])
    mi = jnp.where(pi, mi2[:, EMB:], mi2[:, :EMB])

    cdims = (((1,), (1,)), ((), ()))
    f32 = jnp.float32
    bf = jnp.bfloat16
    gmf = eu * ei
    h = lax.dot_general(mu.astype(bf), w1a[...].astype(bf), cdims,
                        preferred_element_type=f32)
    h = h + lax.dot_general(mi.astype(bf), w1b[...].astype(bf), cdims,
                            preferred_element_type=f32)
    h = jnp.maximum(h + b1[...], 0.0)
    h = jnp.maximum(lax.dot_general(h.astype(bf), w2[...].astype(bf), cdims,
                                    preferred_element_type=f32) + b2[...], 0.0)
    h = jnp.maximum(lax.dot_general(h.astype(bf), w3[...].astype(bf), cdims,
                                    preferred_element_type=f32) + b3[...], 0.0)
    pred = jnp.sum(gmf * wg[...], axis=1) + jnp.sum(h * wh[...], axis=1)
    out[...] = pred + bp[0]


def _full(shape):
    nd = len(shape)
    return pl.BlockSpec(shape, lambda i: (0,) * nd)


def kernel(user, item, Ug, Ig, Um, Im, W1, b1, W2, b2, W3, b3, Wp, bp):
    user = user.astype(jnp.int32)
    item = item.astype(jnp.int32)

    eu2 = _sc_gather(user, _repack(Ug.T))
    ei2 = _sc_gather(item, _repack(Ig.T))
    mu2 = _sc_gather(user, _repack(Um.T))
    mi2 = _sc_gather(item, _repack(Im.T))

    w1a = W1[:, :EMB]
    w1b = W1[:, EMB:]
    h1 = W1.shape[0]
    h2 = W2.shape[0]
    h3 = W3.shape[0]
    wg = Wp[:, :EMB]
    wh = Wp[:, EMB:]

    grid = BATCH // _BB
    pair_spec = pl.BlockSpec((_BB, 2 * EMB), lambda i: (i, 0))
    idx_spec = pl.BlockSpec((_BB, 1), lambda i: (i, 0))
    out = pl.pallas_call(
        _mlp_body,
        grid=(grid,),
        in_specs=[
            idx_spec, idx_spec,
            pair_spec, pair_spec, pair_spec, pair_spec,
            _full((h1, EMB)), _full((h1, EMB)), _full((1, h1)),
            _full((h2, h1)), _full((1, h2)),
            _full((h3, h2)), _full((1, h3)),
            _full((1, EMB)), _full((1, h3)), _full((1,)),
        ],
        out_specs=pl.BlockSpec((_BB,), lambda i: (i,)),
        out_shape=jax.ShapeDtypeStruct((BATCH,), jnp.float32),
    )(user.reshape(BATCH, 1), item.reshape(BATCH, 1),
      eu2, ei2, mu2, mi2, w1a, w1b, b1.reshape(1, h1), W2,
      b2.reshape(1, h2), W3, b3.reshape(1, h3), wg, wh, bp)
    return out


# speedup vs baseline: 3.2281x; 1.2331x over previous
import jax
import jax.numpy as jnp
from jax import lax
from jax.experimental import pallas as pl
from jax.experimental.pallas import tpu as pltpu
from jax.experimental.pallas import tpu_sc as plsc

BATCH = 16384
EMB = 64
NROWS = 1000000
_RCH = 32768
_NCH = (NROWS + _RCH - 1) // _RCH
_LOG_RCH = _RCH.bit_length() - 1
_HMASK = _RCH // 2 - 1

_info = plsc.get_sparse_core_info()
_NC, _NS = _info.num_cores, _info.num_subcores
_NW = _NC * _NS
_BPW = BATCH // _NW


def _repack_body(src, dst):
    y = src[...].astype(jnp.bfloat16).T.astype(jnp.float32)
    dst[...] = jnp.concatenate([y[: _RCH // 2], y[_RCH // 2:]], axis=1)


_repack = pl.pallas_call(
    _repack_body,
    grid=(_NCH,),
    in_specs=[pl.BlockSpec((EMB, _RCH), lambda j: (0, j))],
    out_specs=pl.BlockSpec((_RCH // 2, 2 * EMB), lambda j: (j, 0)),
    out_shape=jax.ShapeDtypeStruct((_NCH * _RCH // 2, 2 * EMB), jnp.float32),
)


def _gather_body(idx_hbm, tbl, out_hbm, idx_v, idxh, vals, sem):
    wid = lax.axis_index("s") * _NC + lax.axis_index("c")
    base = wid * _BPW
    pltpu.sync_copy(idx_hbm.at[pl.ds(base, _BPW)], idx_v)

    def grp(g, c):
        r = idx_v[pl.ds(16 * g, 16)]
        idxh[pl.ds(16 * g, 16)] = (
            (r >> _LOG_RCH) << (_LOG_RCH - 1)) + (r & _HMASK)
        return c

    lax.fori_loop(0, _BPW // 16, grp, 0)
    pltpu.async_copy(tbl.at[idxh], vals, sem).wait()
    pltpu.sync_copy(vals, out_hbm.at[pl.ds(base, _BPW)])


_sc_gather = pl.kernel(
    _gather_body,
    out_type=jax.ShapeDtypeStruct((BATCH, 2 * EMB), jnp.float32),
    mesh=plsc.VectorSubcoreMesh(core_axis_name="c", subcore_axis_name="s"),
    scratch_types=[
        pltpu.VMEM((_BPW,), jnp.int32),
        pltpu.VMEM((_BPW,), jnp.int32),
        pltpu.VMEM((_BPW, 2 * EMB), jnp.float32),
        pltpu.SemaphoreType.DMA,
    ],
    compiler_params=pltpu.CompilerParams(use_tc_tiling_on_sc=False),
)


_BB = 2048


def _mlp_body(u2, i2, eu2, ei2, mu2, mi2,
              w1a, w1b, b1, w2, b2, w3, b3, wg, wh, bp, out):
    pu = ((u2[...] >> (_LOG_RCH - 1)) & 1) == 1
    pi = ((i2[...] >> (_LOG_RCH - 1)) & 1) == 1
    eu = jnp.where(pu, eu2[:, EMB:], eu2[:, :EMB])
    mu = jnp.where(pu, mu2[:, EMB:], mu2[:, :EMB])
    ei = jnp.where(pi, ei2[:, EMB:], ei2[:, :EMB])
    mi = jnp.where(pi, mi2[:, EMB:], mi2[:, :EMB])

    cdims = (((1,), (1,)), ((), ()))
    f32 = jnp.float32
    bf = jnp.bfloat16
    gmf = eu * ei
    h = lax.dot_general(mu.astype(bf), w1a[...].astype(bf), cdims,
                        preferred_element_type=f32)
    h = h + lax.dot_general(mi.astype(bf), w1b[...].astype(bf), cdims,
                            preferred_element_type=f32)
    h = jnp.maximum(h + b1[...], 0.0)
    h = jnp.maximum(lax.dot_general(h.astype(bf), w2[...].astype(bf), cdims,
                                    preferred_element_type=f32) + b2[...], 0.0)
    h = jnp.maximum(lax.dot_general(h.astype(bf), w3[...].astype(bf), cdims,
                                    preferred_element_type=f32) + b3[...], 0.0)
    pred = jnp.sum(gmf * wg[...], axis=1) + jnp.sum(h * wh[...], axis=1)
    out[...] = pred + bp[0]


def _full(shape):
    nd = len(shape)
    return pl.BlockSpec(shape, lambda i: (0,) * nd)


def kernel(user, item, Ug, Ig, Um, Im, W1, b1, W2, b2, W3, b3, Wp, bp):
    user = user.astype(jnp.int32)
    item = item.astype(jnp.int32)

    eu2 = _sc_gather(user, _repack(Ug.T))
    ei2 = _sc_gather(item, _repack(Ig.T))
    mu2 = _sc_gather(user, _repack(Um.T))
    mi2 = _sc_gather(item, _repack(Im.T))

    w1a = W1[:, :EMB]
    w1b = W1[:, EMB:]
    h1 = W1.shape[0]
    h2 = W2.shape[0]
    h3 = W3.shape[0]
    wg = Wp[:, :EMB]
    wh = Wp[:, EMB:]

    grid = BATCH // _BB
    pair_spec = pl.BlockSpec((_BB, 2 * EMB), lambda i: (i, 0))
    idx_spec = pl.BlockSpec((_BB, 1), lambda i: (i, 0))
    out = pl.pallas_call(
        _mlp_body,
        grid=(grid,),
        in_specs=[
            idx_spec, idx_spec,
            pair_spec, pair_spec, pair_spec, pair_spec,
            _full((h1, EMB)), _full((h1, EMB)), _full((1, h1)),
            _full((h2, h1)), _full((1, h2)),
            _full((h3, h2)), _full((1, h3)),
            _full((1, EMB)), _full((1, h3)), _full((1,)),
        ],
        out_specs=pl.BlockSpec((_BB,), lambda i: (i,)),
        out_shape=jax.ShapeDtypeStruct((BATCH,), jnp.float32),
    )(user.reshape(BATCH, 1), item.reshape(BATCH, 1),
      eu2, ei2, mu2, mi2, w1a, w1b, b1.reshape(1, h1), W2,
      b2.reshape(1, h2), W3, b3.reshape(1, h3), wg, wh, bp)
    return out
